# baseline (device time: 83886 ns/iter reference)
import numpy as np

import jax
import jax.numpy as jnp
from jax import lax
from jax.experimental import pallas as pl
from jax.experimental.pallas import tpu as pltpu

N_RING = 16
CYCLE = [
    (0, 0), (0, 1), (0, 2), (0, 3),
    (1, 3), (2, 3), (3, 3), (3, 2),
    (2, 2), (1, 2), (1, 1), (2, 1),
    (3, 1), (3, 0), (2, 0), (1, 0),
]
RING_POS = np.zeros(N_RING, np.int32)
for _i, (_y, _z) in enumerate(CYCLE):
    RING_POS[_y * 4 + _z] = _i
CYCLE_Y = np.array([c[0] for c in CYCLE], np.int32)
CYCLE_Z = np.array([c[1] for c in CYCLE], np.int32)

M = 2048 // 2
D = 1024
F = 4096
CHUNK = F // N_RING
HALF = D // 2

CW_HOPS = N_RING // 2
CCW_HOPS = N_RING // 2 - 1


def kernel(x, dy):
    my_x = lax.axis_index("x").astype(jnp.int32)
    my_y = lax.axis_index("y").astype(jnp.int32)
    my_z = lax.axis_index("z").astype(jnp.int32)

    p = jnp.asarray(RING_POS)[my_y * 4 + my_z]
    nxt = (p + 1) % N_RING
    prv = (p + N_RING - 1) % N_RING
    cyc_y = jnp.asarray(CYCLE_Y)
    cyc_z = jnp.asarray(CYCLE_Z)
    meta = jnp.stack(
        [p, my_x, my_y, my_z, cyc_y[nxt], cyc_z[nxt], cyc_y[prv], cyc_z[prv]]
    ).astype(jnp.int32)

    dy_chunk = lax.dynamic_slice(dy, (0, p * CHUNK), (M, CHUNK))

    def body(meta_ref, x_ref, dyc_ref, out_ref,
             part_scr, xrecv_scr,
             x_send_sem, x_recv_sem,
             cw_send_sems, cw_recv_sems,
             ccw_send_sems, ccw_recv_sems):
        pp = meta_ref[0]
        myx = meta_ref[1]
        myy = meta_ref[2]
        myz = meta_ref[3]
        ny, nz = meta_ref[4], meta_ref[5]
        py, pz = meta_ref[6], meta_ref[7]

        barrier = pltpu.get_barrier_semaphore()
        for tgt in [(1 - myx, myy, myz), (myx, ny, nz), (myx, py, pz)]:
            pl.semaphore_signal(
                barrier, inc=1, device_id=tgt,
                device_id_type=pl.DeviceIdType.MESH,
            )
        pl.semaphore_wait(barrier, 3)

        part_scr[...] = lax.dot_general(
            x_ref[...], dyc_ref[...],
            dimension_numbers=(((0,), (0,)), ((), ())),
            preferred_element_type=jnp.float32,
        )

        my_off = myx * HALF
        other_off = (1 - myx) * HALF
        xchg = pltpu.make_async_remote_copy(
            src_ref=part_scr.at[pl.ds(other_off, HALF), :],
            dst_ref=xrecv_scr,
            send_sem=x_send_sem,
            recv_sem=x_recv_sem,
            device_id=(1 - myx, myy, myz),
            device_id_type=pl.DeviceIdType.MESH,
        )
        xchg.start()
        xchg.wait()

        out_ref[:, pl.ds(pp * CHUNK, CHUNK)] = (
            part_scr[pl.ds(my_off, HALF), :] + xrecv_scr[...]
        )

        for h in range(CW_HOPS):
            o_cw = (pp + N_RING - h) % N_RING
            cw = pltpu.make_async_remote_copy(
                src_ref=out_ref.at[:, pl.ds(o_cw * CHUNK, CHUNK)],
                dst_ref=out_ref.at[:, pl.ds(o_cw * CHUNK, CHUNK)],
                send_sem=cw_send_sems.at[h],
                recv_sem=cw_recv_sems.at[h],
                device_id=(myx, ny, nz),
                device_id_type=pl.DeviceIdType.MESH,
            )
            cw.start()
            if h < CCW_HOPS:
                o_ccw = (pp + h) % N_RING
                ccw = pltpu.make_async_remote_copy(
                    src_ref=out_ref.at[:, pl.ds(o_ccw * CHUNK, CHUNK)],
                    dst_ref=out_ref.at[:, pl.ds(o_ccw * CHUNK, CHUNK)],
                    send_sem=ccw_send_sems.at[h],
                    recv_sem=ccw_recv_sems.at[h],
                    device_id=(myx, py, pz),
                    device_id_type=pl.DeviceIdType.MESH,
                )
                ccw.start()
            cw.wait()
            if h < CCW_HOPS:
                ccw.wait()

    return pl.pallas_call(
        body,
        out_shape=jax.ShapeDtypeStruct((HALF, F), jnp.float32),
        in_specs=[
            pl.BlockSpec(memory_space=pltpu.SMEM),
            pl.BlockSpec(memory_space=pltpu.VMEM),
            pl.BlockSpec(memory_space=pltpu.VMEM),
        ],
        out_specs=pl.BlockSpec(memory_space=pltpu.VMEM),
        scratch_shapes=[
            pltpu.VMEM((M, CHUNK), jnp.float32),
            pltpu.VMEM((HALF, CHUNK), jnp.float32),
            pltpu.SemaphoreType.DMA,
            pltpu.SemaphoreType.DMA,
            pltpu.SemaphoreType.DMA((CW_HOPS,)),
            pltpu.SemaphoreType.DMA((CW_HOPS,)),
            pltpu.SemaphoreType.DMA((CCW_HOPS,)),
            pltpu.SemaphoreType.DMA((CCW_HOPS,)),
        ],
        compiler_params=pltpu.CompilerParams(collective_id=0),
    )(meta, x, dy_chunk)


# device time: 70973 ns/iter; 1.1819x vs baseline; 1.1819x over previous
import numpy as np

import jax
import jax.numpy as jnp
from jax import lax
from jax.experimental import pallas as pl
from jax.experimental.pallas import tpu as pltpu

N_RING = 16
CYCLE = [
    (0, 0), (0, 1), (0, 2), (0, 3),
    (1, 3), (2, 3), (3, 3), (3, 2),
    (2, 2), (1, 2), (1, 1), (2, 1),
    (3, 1), (3, 0), (2, 0), (1, 0),
]
RING_POS = np.zeros(N_RING, np.int32)
for _i, (_y, _z) in enumerate(CYCLE):
    RING_POS[_y * 4 + _z] = _i
CYCLE_Y = np.array([c[0] for c in CYCLE], np.int32)
CYCLE_Z = np.array([c[1] for c in CYCLE], np.int32)

M = 2048 // 2
D = 1024
F = 4096
CHUNK = F // N_RING
HALF = D // 2

CW_HOPS = N_RING // 2
CCW_HOPS = N_RING // 2 - 1
S = 2
SUB = CHUNK // S


def kernel(x, dy):
    my_x = lax.axis_index("x").astype(jnp.int32)
    my_y = lax.axis_index("y").astype(jnp.int32)
    my_z = lax.axis_index("z").astype(jnp.int32)

    p = jnp.asarray(RING_POS)[my_y * 4 + my_z]
    nxt = (p + 1) % N_RING
    prv = (p + N_RING - 1) % N_RING
    cyc_y = jnp.asarray(CYCLE_Y)
    cyc_z = jnp.asarray(CYCLE_Z)
    meta = jnp.stack(
        [p, my_x, my_y, my_z, cyc_y[nxt], cyc_z[nxt], cyc_y[prv], cyc_z[prv]]
    ).astype(jnp.int32)

    dy_chunk = lax.dynamic_slice(dy, (0, p * CHUNK), (M, CHUNK))

    def body(meta_ref, x_ref, dyc_ref, out_ref,
             part_scr, xrecv_scr,
             x_send_sem, x_recv_sem,
             cw_send_sems, cw_recv_sems,
             ccw_send_sems, ccw_recv_sems):
        pp = meta_ref[0]
        myx = meta_ref[1]
        myy = meta_ref[2]
        myz = meta_ref[3]
        ny, nz = meta_ref[4], meta_ref[5]
        py, pz = meta_ref[6], meta_ref[7]

        barrier = pltpu.get_barrier_semaphore()
        for tgt in [(1 - myx, myy, myz), (myx, ny, nz), (myx, py, pz)]:
            pl.semaphore_signal(
                barrier, inc=1, device_id=tgt,
                device_id_type=pl.DeviceIdType.MESH,
            )
        pl.semaphore_wait(barrier, 3)

        my_off = myx * HALF
        other_off = (1 - myx) * HALF
        part_scr[pl.ds(other_off, HALF), :] = lax.dot_general(
            x_ref[:, pl.ds(other_off, HALF)], dyc_ref[...],
            dimension_numbers=(((0,), (0,)), ((), ())),
            preferred_element_type=jnp.float32,
        )
        xchg = pltpu.make_async_remote_copy(
            src_ref=part_scr.at[pl.ds(other_off, HALF), :],
            dst_ref=xrecv_scr,
            send_sem=x_send_sem,
            recv_sem=x_recv_sem,
            device_id=(1 - myx, myy, myz),
            device_id_type=pl.DeviceIdType.MESH,
        )
        xchg.start()
        part_scr[pl.ds(my_off, HALF), :] = lax.dot_general(
            x_ref[:, pl.ds(my_off, HALF)], dyc_ref[...],
            dimension_numbers=(((0,), (0,)), ((), ())),
            preferred_element_type=jnp.float32,
        )
        xchg.wait()

        out_ref[:, pl.ds(pp * CHUNK, CHUNK)] = (
            part_scr[pl.ds(my_off, HALF), :] + xrecv_scr[...]
        )

        def cw_copy(h, s):
            col = ((pp + N_RING - h) % N_RING) * CHUNK + s * SUB
            return pltpu.make_async_remote_copy(
                src_ref=out_ref.at[:, pl.ds(col, SUB)],
                dst_ref=out_ref.at[:, pl.ds(col, SUB)],
                send_sem=cw_send_sems.at[h, s],
                recv_sem=cw_recv_sems.at[h, s],
                device_id=(myx, ny, nz),
                device_id_type=pl.DeviceIdType.MESH,
            )

        def ccw_copy(h, s):
            col = ((pp + h) % N_RING) * CHUNK + s * SUB
            return pltpu.make_async_remote_copy(
                src_ref=out_ref.at[:, pl.ds(col, SUB)],
                dst_ref=out_ref.at[:, pl.ds(col, SUB)],
                send_sem=ccw_send_sems.at[h, s],
                recv_sem=ccw_recv_sems.at[h, s],
                device_id=(myx, py, pz),
                device_id_type=pl.DeviceIdType.MESH,
            )

        cw_d = [[cw_copy(h, s) for s in range(S)] for h in range(CW_HOPS)]
        ccw_d = [[ccw_copy(h, s) for s in range(S)] for h in range(CCW_HOPS)]

        for s in range(S):
            cw_d[0][s].start()
            ccw_d[0][s].start()
        for h in range(1, CW_HOPS):
            for s in range(S):
                cw_d[h - 1][s].wait_recv()
                cw_d[h][s].start()
            if h < CCW_HOPS:
                for s in range(S):
                    ccw_d[h - 1][s].wait_recv()
                    ccw_d[h][s].start()
        for s in range(S):
            cw_d[CW_HOPS - 1][s].wait_recv()
            ccw_d[CCW_HOPS - 1][s].wait_recv()
        for h in range(CW_HOPS):
            for s in range(S):
                cw_d[h][s].wait_send()
        for h in range(CCW_HOPS):
            for s in range(S):
                ccw_d[h][s].wait_send()

    return pl.pallas_call(
        body,
        out_shape=jax.ShapeDtypeStruct((HALF, F), jnp.float32),
        in_specs=[
            pl.BlockSpec(memory_space=pltpu.SMEM),
            pl.BlockSpec(memory_space=pltpu.VMEM),
            pl.BlockSpec(memory_space=pltpu.VMEM),
        ],
        out_specs=pl.BlockSpec(memory_space=pltpu.VMEM),
        scratch_shapes=[
            pltpu.VMEM((M, CHUNK), jnp.float32),
            pltpu.VMEM((HALF, CHUNK), jnp.float32),
            pltpu.SemaphoreType.DMA,
            pltpu.SemaphoreType.DMA,
            pltpu.SemaphoreType.DMA((CW_HOPS, S)),
            pltpu.SemaphoreType.DMA((CW_HOPS, S)),
            pltpu.SemaphoreType.DMA((CCW_HOPS, S)),
            pltpu.SemaphoreType.DMA((CCW_HOPS, S)),
        ],
        compiler_params=pltpu.CompilerParams(collective_id=0),
    )(meta, x, dy_chunk)


# device time: 60993 ns/iter; 1.3753x vs baseline; 1.1636x over previous
import numpy as np

import jax
import jax.numpy as jnp
from jax import lax
from jax.experimental import pallas as pl
from jax.experimental.pallas import tpu as pltpu

N_RING = 16
CYCLE = [
    (0, 0), (0, 1), (0, 2), (0, 3),
    (1, 3), (2, 3), (3, 3), (3, 2),
    (2, 2), (1, 2), (1, 1), (2, 1),
    (3, 1), (3, 0), (2, 0), (1, 0),
]
RING_POS = [0] * N_RING
for _i, (_y, _z) in enumerate(CYCLE):
    RING_POS[_y * 4 + _z] = _i
CYCLE_Y = [c[0] for c in CYCLE]
CYCLE_Z = [c[1] for c in CYCLE]

M = 2048 // 2
D = 1024
F = 4096
CHUNK = F // N_RING
HALF = D // 2

S = 2
SUB = CHUNK // S
CW_HOPS = (N_RING // 2, N_RING // 2 - 1)
CCW_HOPS = (N_RING // 2 - 1, N_RING // 2)
MAX_HOPS = N_RING // 2


def _lut(idx, table):
    v = jnp.int32(table[0])
    for i in range(1, len(table)):
        v = jnp.where(idx == i, jnp.int32(table[i]), v)
    return v


def kernel(x, dy):
    def body(x_ref, dy_ref, out_ref,
             dyc_scr, part_scr, xrecv_scr,
             dy_sem, x_send_sems, x_recv_sems,
             cw_send_sems, cw_recv_sems,
             ccw_send_sems, ccw_recv_sems):
        myx = lax.axis_index("x").astype(jnp.int32)
        myy = lax.axis_index("y").astype(jnp.int32)
        myz = lax.axis_index("z").astype(jnp.int32)
        pp = _lut(myy * 4 + myz, RING_POS)
        nxt = lax.rem(pp + 1, N_RING)
        prv = lax.rem(pp + N_RING - 1, N_RING)
        ny, nz = _lut(nxt, CYCLE_Y), _lut(nxt, CYCLE_Z)
        py, pz = _lut(prv, CYCLE_Y), _lut(prv, CYCLE_Z)

        dy_dma = pltpu.make_async_copy(
            dy_ref.at[:, pl.ds(pp * CHUNK, CHUNK)], dyc_scr, dy_sem
        )
        dy_dma.start()

        barrier = pltpu.get_barrier_semaphore()
        for tgt in [(1 - myx, myy, myz), (myx, ny, nz), (myx, py, pz)]:
            pl.semaphore_signal(
                barrier, inc=1, device_id=tgt,
                device_id_type=pl.DeviceIdType.MESH,
            )
        pl.semaphore_wait(barrier, 3)
        dy_dma.wait()

        my_off = myx * HALF
        other_off = (1 - myx) * HALF
        part_scr[pl.ds(other_off, HALF), :] = lax.dot_general(
            x_ref[:, pl.ds(other_off, HALF)], dyc_scr[...],
            dimension_numbers=(((0,), (0,)), ((), ())),
            preferred_element_type=jnp.float32,
        )
        xchg = [
            pltpu.make_async_remote_copy(
                src_ref=part_scr.at[pl.ds(other_off, HALF),
                                    pl.ds(s * SUB, SUB)],
                dst_ref=xrecv_scr.at[:, pl.ds(s * SUB, SUB)],
                send_sem=x_send_sems.at[s],
                recv_sem=x_recv_sems.at[s],
                device_id=(1 - myx, myy, myz),
                device_id_type=pl.DeviceIdType.MESH,
            )
            for s in range(S)
        ]
        for s in range(S):
            xchg[s].start()
        part_scr[pl.ds(my_off, HALF), :] = lax.dot_general(
            x_ref[:, pl.ds(my_off, HALF)], dyc_scr[...],
            dimension_numbers=(((0,), (0,)), ((), ())),
            preferred_element_type=jnp.float32,
        )

        def cw_copy(h, s):
            col = lax.rem(pp + N_RING - h, N_RING) * CHUNK + s * SUB
            return pltpu.make_async_remote_copy(
                src_ref=out_ref.at[:, pl.ds(col, SUB)],
                dst_ref=out_ref.at[:, pl.ds(col, SUB)],
                send_sem=cw_send_sems.at[h, s],
                recv_sem=cw_recv_sems.at[h, s],
                device_id=(myx, ny, nz),
                device_id_type=pl.DeviceIdType.MESH,
            )

        def ccw_copy(h, s):
            col = lax.rem(pp + h, N_RING) * CHUNK + s * SUB
            return pltpu.make_async_remote_copy(
                src_ref=out_ref.at[:, pl.ds(col, SUB)],
                dst_ref=out_ref.at[:, pl.ds(col, SUB)],
                send_sem=ccw_send_sems.at[h, s],
                recv_sem=ccw_recv_sems.at[h, s],
                device_id=(myx, py, pz),
                device_id_type=pl.DeviceIdType.MESH,
            )

        cw_d = [[cw_copy(h, s) for s in range(S)] for h in range(MAX_HOPS)]
        ccw_d = [[ccw_copy(h, s) for s in range(S)] for h in range(MAX_HOPS)]

        for s in range(S):
            xchg[s].wait()
            out_ref[:, pl.ds(pp * CHUNK + s * SUB, SUB)] = (
                part_scr[pl.ds(my_off, HALF), pl.ds(s * SUB, SUB)]
                + xrecv_scr[:, pl.ds(s * SUB, SUB)]
            )
            cw_d[0][s].start()
            ccw_d[0][s].start()

        for h in range(1, MAX_HOPS):
            for s in range(S):
                if h < CW_HOPS[s]:
                    cw_d[h - 1][s].wait_recv()
                    cw_d[h][s].start()
                if h < CCW_HOPS[s]:
                    ccw_d[h - 1][s].wait_recv()
                    ccw_d[h][s].start()
        for s in range(S):
            cw_d[CW_HOPS[s] - 1][s].wait_recv()
            ccw_d[CCW_HOPS[s] - 1][s].wait_recv()
        for s in range(S):
            for h in range(CW_HOPS[s]):
                cw_d[h][s].wait_send()
            for h in range(CCW_HOPS[s]):
                ccw_d[h][s].wait_send()

    return pl.pallas_call(
        body,
        out_shape=jax.ShapeDtypeStruct((HALF, F), jnp.float32),
        in_specs=[
            pl.BlockSpec(memory_space=pltpu.VMEM),
            pl.BlockSpec(memory_space=pl.ANY),
        ],
        out_specs=pl.BlockSpec(memory_space=pltpu.VMEM),
        scratch_shapes=[
            pltpu.VMEM((M, CHUNK), jnp.float32),
            pltpu.VMEM((M, CHUNK), jnp.float32),
            pltpu.VMEM((HALF, CHUNK), jnp.float32),
            pltpu.SemaphoreType.DMA,
            pltpu.SemaphoreType.DMA((S,)),
            pltpu.SemaphoreType.DMA((S,)),
            pltpu.SemaphoreType.DMA((MAX_HOPS, S)),
            pltpu.SemaphoreType.DMA((MAX_HOPS, S)),
            pltpu.SemaphoreType.DMA((MAX_HOPS, S)),
            pltpu.SemaphoreType.DMA((MAX_HOPS, S)),
        ],
        compiler_params=pltpu.CompilerParams(collective_id=0),
    )(x, dy)
